# Initial kernel scaffold; baseline (speedup 1.0000x reference)
import jax
import jax.numpy as jnp
from jax.experimental import pallas as pl


def kernel(prediction):
    raise NotImplementedError("write your pallas kernel here")


# speedup vs baseline: 26.2607x; 26.2607x over previous
import functools

import jax
import jax.numpy as jnp
from jax import lax
from jax.experimental import pallas as pl
from jax.experimental.pallas import tpu as pltpu
from jax.experimental.pallas import tpu_sc as plsc

CONF_THRES = 0.9
IOU_THRES = 0.45
MAX_WH = 4096.0
MAX_DET = 300
PRE_NMS = 2048

B = 4
N = 20000
NC = 80
NFLAT = N * NC
NTILES = 32
TPI = NTILES // B
RANGE = NFLAT // TPI
CHUNK = 8000
CT = 2048
SORT_N = TPI * CT
OUTW = MAX_DET * 6


def _stage_a_body(pred_ref, scores_ref, boxes_ref):
    x = pred_ref[0]
    xy = x[:, 0:2]
    wh = x[:, 2:4]
    obj = x[:, 4:5]
    cls = x[:, 5:85]
    conf = cls * obj
    valid = (obj > CONF_THRES) & (conf > CONF_THRES)
    scores_ref[0] = jnp.where(valid, conf, -1.0)
    half = wh / 2.0
    boxes_ref[0] = jnp.concatenate([xy - half, xy + half], axis=-1)


def _stage_a(prediction):
    rows = 4000
    grid = (B, N // rows)
    return pl.pallas_call(
        _stage_a_body,
        grid=grid,
        in_specs=[pl.BlockSpec((1, rows, 85), lambda i, j: (i, j, 0))],
        out_specs=[
            pl.BlockSpec((1, rows, NC), lambda i, j: (i, j, 0)),
            pl.BlockSpec((1, rows, 4), lambda i, j: (i, j, 0)),
        ],
        out_shape=[
            jax.ShapeDtypeStruct((B, N, NC), jnp.float32),
            jax.ShapeDtypeStruct((B, N, 4), jnp.float32),
        ],
    )(prediction)


def _stage_b_kernel(scores_hbm, vals_hbm, idx_hbm, chunk_v, valbuf, idxbuf):
    wid = lax.axis_index("s") * 2 + lax.axis_index("c")
    img = wid // TPI
    sub = wid % TPI
    base = sub * RANGE

    def init_body(i, _):
        valbuf[pl.ds(i * 16, 16)] = jnp.full((16,), -1.0, jnp.float32)
        idxbuf[pl.ds(i * 16, 16)] = jnp.zeros((16,), jnp.int32)
        return 0

    lax.fori_loop(0, CT // 16, init_body, 0)

    lane = lax.iota(jnp.int32, 16)

    def chunk_body(ci, cnt):
        pltpu.sync_copy(
            scores_hbm.at[pl.ds(img * NFLAT + base + ci * CHUNK, CHUNK)],
            chunk_v)

        def vec_body(vi, cnt_in):
            vals = chunk_v[pl.ds(vi * 16, 16)]
            m = vals > 0.0
            gidx = lane + (base + ci * CHUNK + vi * 16)
            mi = m.astype(jnp.int32)
            pos = jnp.minimum(cnt_in + plsc.cumsum(mi) - mi, CT - 1)
            plsc.store_scatter(valbuf, [pos], vals, mask=m)
            plsc.store_scatter(idxbuf, [pos], gidx, mask=m)
            return cnt_in + jnp.sum(mi)

        return lax.fori_loop(0, CHUNK // 16, vec_body, cnt)

    lax.fori_loop(0, RANGE // CHUNK, chunk_body, jnp.int32(0))

    pltpu.sync_copy(valbuf, vals_hbm.at[pl.ds(img * SORT_N + sub * CT, CT)])
    pltpu.sync_copy(idxbuf, idx_hbm.at[pl.ds(img * SORT_N + sub * CT, CT)])


def _stage_b(scores_flat):
    mesh = plsc.VectorSubcoreMesh(core_axis_name="c", subcore_axis_name="s")
    k = functools.partial(
        pl.kernel,
        mesh=mesh,
        compiler_params=pltpu.CompilerParams(needs_layout_passes=False),
        out_type=[
            jax.ShapeDtypeStruct((B * SORT_N,), jnp.float32),
            jax.ShapeDtypeStruct((B * SORT_N,), jnp.int32),
        ],
        scratch_types=[
            pltpu.VMEM((CHUNK,), jnp.float32),
            pltpu.VMEM((CT,), jnp.float32),
            pltpu.VMEM((CT,), jnp.int32),
        ],
    )(_stage_b_kernel)
    vals, idxs = k(scores_flat.reshape(B * NFLAT))
    return vals.reshape(B, SORT_N), idxs.reshape(B, SORT_N)


def _roll(x, shift):
    return jnp.concatenate([x[..., -shift % 128:], x[..., : -shift % 128]],
                           axis=-1)


def _stage_c_body(val_ref, idx_ref, ts_ref, bi_ref, cf_ref):
    s = val_ref[...]
    ix = idx_ref[...]
    r_io = lax.broadcasted_iota(jnp.int32, (B, 128, 128), 1)
    c_io = lax.broadcasted_iota(jnp.int32, (B, 128, 128), 2)

    n_total = 128 * 128
    k = 2
    while k <= n_total:
        j = k // 2
        while j >= 1:
            if j >= 128:
                rj = j // 128
                sh = (B, 128 // (2 * rj), 2, rj, 128)
                s4 = s.reshape(sh)
                i4 = ix.reshape(sh)
                s_p = jnp.concatenate([s4[:, :, 1:2], s4[:, :, 0:1]],
                                      axis=2).reshape(B, 128, 128)
                i_p = jnp.concatenate([i4[:, :, 1:2], i4[:, :, 0:1]],
                                      axis=2).reshape(B, 128, 128)
                am_hi = (r_io & rj) != 0
            else:
                s_m = _roll(s, -j)
                s_q = _roll(s, j)
                i_m = _roll(ix, -j)
                i_q = _roll(ix, j)
                lane_lo = (c_io & j) == 0
                s_p = jnp.where(lane_lo, s_m, s_q)
                i_p = jnp.where(lane_lo, i_m, i_q)
                am_hi = ~lane_lo
            if k >= 128:
                asc = (r_io & (k // 128)) == 0
            else:
                asc = (c_io & k) == 0
            take_high = asc != am_hi
            mine_high = (s > s_p) | ((s == s_p) & (ix < i_p))
            sel_mine = ~(take_high ^ mine_high)
            s = jnp.where(sel_mine, s, s_p)
            ix = jnp.where(sel_mine, ix, i_p)
            j //= 2
        k *= 2

    ts = s[:, :16, :]
    ti = ix[:, :16, :]
    ts_ref[...] = ts
    ti_f = ti.astype(jnp.float32)
    bi_f = jnp.floor((ti_f + 0.5) * (1.0 / NC))
    bi_ref[...] = bi_f.astype(jnp.int32)
    cf_ref[...] = ti_f - bi_f * float(NC)


def _stage_c(vals, idxs):
    v3 = vals.reshape(B, 128, 128)
    i3 = idxs.reshape(B, 128, 128)
    ts, bi, cf = pl.pallas_call(
        _stage_c_body,
        out_shape=[
            jax.ShapeDtypeStruct((B, 16, 128), jnp.float32),
            jax.ShapeDtypeStruct((B, 16, 128), jnp.int32),
            jax.ShapeDtypeStruct((B, 16, 128), jnp.float32),
        ],
    )(v3, i3)
    return (ts.reshape(B, PRE_NMS), bi.reshape(B, PRE_NMS),
            cf.reshape(B, PRE_NMS))


def _stage_d_kernel(boxes_hbm, bidx_hbm, out_hbm, table_v, idx_v, ob):
    wid = lax.axis_index("s") * 2 + lax.axis_index("c")
    img = wid // TPI
    sub = wid % TPI
    per = PRE_NMS // TPI

    pltpu.sync_copy(boxes_hbm.at[pl.ds(img * N * 4, N * 4)], table_v)
    pltpu.sync_copy(bidx_hbm.at[pl.ds(img * PRE_NMS + sub * per, per)], idx_v)

    def body(g, _):
        iv = idx_v[pl.ds(g * 16, 16)]
        for c in range(4):
            v = plsc.load_gather(table_v, [iv * 4 + c])
            ob[c, pl.ds(g * 16, 16)] = v
        return 0

    lax.fori_loop(0, per // 16, body, 0)
    for c in range(4):
        pltpu.sync_copy(
            ob.at[c],
            out_hbm.at[pl.ds(img * 4 * PRE_NMS + c * PRE_NMS + sub * per,
                             per)])


def _stage_d(boxes_flat, bidx):
    mesh = plsc.VectorSubcoreMesh(core_axis_name="c", subcore_axis_name="s")
    per = PRE_NMS // TPI
    k = functools.partial(
        pl.kernel,
        mesh=mesh,
        compiler_params=pltpu.CompilerParams(needs_layout_passes=False),
        out_type=jax.ShapeDtypeStruct((B * 4 * PRE_NMS,), jnp.float32),
        scratch_types=[
            pltpu.VMEM((N * 4,), jnp.float32),
            pltpu.VMEM((per,), jnp.int32),
            pltpu.VMEM((4, per), jnp.float32),
        ],
    )(_stage_d_kernel)
    return k(boxes_flat.reshape(B * N * 4),
             bidx.reshape(B * PRE_NMS)).reshape(B, 4, PRE_NMS)


def _stage_e_body(boxp_ref, cls_ref, sc_ref, keep_ref, s_scr):
    cls = cls_ref[...]
    off = cls * MAX_WH
    x1 = boxp_ref[:, 0, :] + off
    y1 = boxp_ref[:, 1, :] + off
    x2 = boxp_ref[:, 2, :] + off
    y2 = boxp_ref[:, 3, :] + off
    area = (x2 - x1) * (y2 - y1)
    valid = jnp.where(sc_ref[...] > 0.0, 1.0, 0.0)

    lane128 = lax.broadcasted_iota(jnp.int32, (1, 128), 1)
    supp = jnp.zeros((B, PRE_NMS), jnp.float32)
    keeps = []
    for b in range(16):
        lo = b * 128
        W = PRE_NMS - lo
        bx1 = x1[:, lo:lo + 128]
        by1 = y1[:, lo:lo + 128]
        bx2 = x2[:, lo:lo + 128]
        by2 = y2[:, lo:lo + 128]
        barea = area[:, lo:lo + 128]
        tx1 = x1[:, lo:]
        ty1 = y1[:, lo:]
        tx2 = x2[:, lo:]
        ty2 = y2[:, lo:]
        tarea = area[:, lo:]
        ltx = jnp.maximum(bx1[:, :, None], tx1[:, None, :])
        lty = jnp.maximum(by1[:, :, None], ty1[:, None, :])
        rbx = jnp.minimum(bx2[:, :, None], tx2[:, None, :])
        rby = jnp.minimum(by2[:, :, None], ty2[:, None, :])
        iw = jnp.clip(rbx - ltx, 0.0, None)
        ih = jnp.clip(rby - lty, 0.0, None)
        inter = iw * ih
        den = barea[:, :, None] + tarea[:, None, :] - inter + 1e-9
        iou = inter / den
        s_scr[:, :, 0:W] = jnp.where(iou > IOU_THRES, 1.0, 0.0)

        valid_b = valid[:, lo:lo + 128]
        supp_b0 = supp[:, lo:lo + 128]

        def body(i8, st):
            supp_b, keep_b = st
            base = pl.multiple_of(i8 * 8, 8)
            s8 = s_scr[:, pl.ds(base, 8), 0:128]
            for r in range(8):
                i = i8 * 8 + r
                e = jnp.where(lane128 == i, 1.0, 0.0)
                r_vec = e * valid_b * (1.0 - supp_b)
                rr = jnp.max(r_vec, axis=-1, keepdims=True)
                srow = s8[:, r, :]
                supp_b = jnp.maximum(supp_b, rr * srow)
                keep_b = jnp.maximum(keep_b, r_vec)
            return supp_b, keep_b

        keep_b0 = jnp.zeros((B, 128), jnp.float32)
        supp_bf, keep_bf = lax.fori_loop(0, 16, body, (supp_b0, keep_b0))
        keeps.append(keep_bf)

        s_tail = s_scr[:, :, 0:W]
        prop = jnp.max(keep_bf[:, :, None] * s_tail, axis=1)
        if lo > 0:
            prop = jnp.concatenate(
                [jnp.zeros((B, lo), jnp.float32), prop], axis=1)
        supp = jnp.maximum(supp, prop)

    keep_ref[...] = jnp.concatenate(keeps, axis=1)


def _stage_e(boxesP, cls_f, top_scores):
    return pl.pallas_call(
        _stage_e_body,
        out_shape=jax.ShapeDtypeStruct((B, PRE_NMS), jnp.float32),
        scratch_shapes=[pltpu.VMEM((B, 128, PRE_NMS), jnp.float32)],
    )(boxesP, cls_f, top_scores)


BUFW = 1808


def _stage_f_kernel(keep_hbm, sc_hbm, cls_hbm, boxp_hbm, out_hbm,
                    keep_v, sc_v, cls_v, box_v, buf):
    wid = lax.axis_index("s") * 2 + lax.axis_index("c")

    @pl.when(wid < B)
    def _():
        img = wid

        def zero_body(i, _):
            buf[pl.ds(i * 16, 16)] = jnp.zeros((16,), jnp.float32)
            return 0

        lax.fori_loop(0, BUFW // 16, zero_body, 0)

        pltpu.sync_copy(keep_hbm.at[pl.ds(img * PRE_NMS, PRE_NMS)], keep_v)
        pltpu.sync_copy(sc_hbm.at[pl.ds(img * PRE_NMS, PRE_NMS)], sc_v)
        pltpu.sync_copy(cls_hbm.at[pl.ds(img * PRE_NMS, PRE_NMS)], cls_v)
        pltpu.sync_copy(boxp_hbm.at[pl.ds(img * 4 * PRE_NMS, 4 * PRE_NMS)],
                        box_v)

        def body(g, cnt):
            km = keep_v[pl.ds(g * 16, 16)] > 0.5
            ki = km.astype(jnp.int32)
            pos = plsc.cumsum(ki) - ki + cnt
            pos = jnp.minimum(pos, MAX_DET)
            mw = km & (pos < MAX_DET)
            base6 = pos * 6
            for f in range(6):
                if f < 4:
                    v = box_v[pl.ds(f * PRE_NMS + g * 16, 16)]
                elif f == 4:
                    v = sc_v[pl.ds(g * 16, 16)]
                else:
                    v = cls_v[pl.ds(g * 16, 16)]
                plsc.store_scatter(buf, [base6 + f], v, mask=mw)
            return cnt + jnp.sum(ki)

        lax.fori_loop(0, PRE_NMS // 16, body, jnp.int32(0))
        pltpu.sync_copy(buf.at[pl.ds(0, OUTW)],
                        out_hbm.at[pl.ds(img * OUTW, OUTW)])


def _stage_f(keep, top_scores, cls_f, boxesP):
    mesh = plsc.VectorSubcoreMesh(core_axis_name="c", subcore_axis_name="s")
    k = functools.partial(
        pl.kernel,
        mesh=mesh,
        compiler_params=pltpu.CompilerParams(needs_layout_passes=False),
        out_type=jax.ShapeDtypeStruct((B * OUTW,), jnp.float32),
        scratch_types=[
            pltpu.VMEM((PRE_NMS,), jnp.float32),
            pltpu.VMEM((PRE_NMS,), jnp.float32),
            pltpu.VMEM((PRE_NMS,), jnp.float32),
            pltpu.VMEM((4 * PRE_NMS,), jnp.float32),
            pltpu.VMEM((BUFW,), jnp.float32),
        ],
    )(_stage_f_kernel)
    return k(keep.reshape(B * PRE_NMS), top_scores.reshape(B * PRE_NMS),
             cls_f.reshape(B * PRE_NMS),
             boxesP.reshape(B * 4 * PRE_NMS)).reshape(B, OUTW)


def kernel(prediction):
    scores, boxes = _stage_a(prediction)
    vals, idxs = _stage_b(scores.reshape(B, NFLAT))
    ts, bi, cf = _stage_c(vals, idxs)
    boxesP = _stage_d(boxes.reshape(B, N * 4), bi)
    keep = _stage_e(boxesP, cf, ts)
    out = _stage_f(keep, ts, cf, boxesP)
    return out.reshape(B, MAX_DET, 6)


# speedup vs baseline: 35.2756x; 1.3433x over previous
import functools

import jax
import jax.numpy as jnp
from jax import lax
from jax.experimental import pallas as pl
from jax.experimental.pallas import tpu as pltpu
from jax.experimental.pallas import tpu_sc as plsc

CONF_THRES = 0.9
IOU_THRES = 0.45
MAX_WH = 4096.0
MAX_DET = 300
PRE_NMS = 2048

B = 4
N = 20000
NC = 80
NFLAT = N * NC
NTILES = 32
TPI = NTILES // B
RANGE = NFLAT // TPI
CHUNK = 8000
CT = 2048
SORT_N = TPI * CT
OUTW = MAX_DET * 6


def _stage_a_body(pred_ref, scores_ref, boxes_ref):
    x = pred_ref[0]
    xy = x[:, 0:2]
    wh = x[:, 2:4]
    obj = x[:, 4:5]
    cls = x[:, 5:85]
    conf = cls * obj
    valid = (obj > CONF_THRES) & (conf > CONF_THRES)
    scores_ref[0] = jnp.where(valid, conf, -1.0)
    half = wh / 2.0
    boxes_ref[0] = jnp.concatenate([xy - half, xy + half], axis=-1)


def _stage_a(prediction):
    rows = 4000
    grid = (B, N // rows)
    return pl.pallas_call(
        _stage_a_body,
        grid=grid,
        in_specs=[pl.BlockSpec((1, rows, 85), lambda i, j: (i, j, 0))],
        out_specs=[
            pl.BlockSpec((1, rows, NC), lambda i, j: (i, j, 0)),
            pl.BlockSpec((1, rows, 4), lambda i, j: (i, j, 0)),
        ],
        out_shape=[
            jax.ShapeDtypeStruct((B, N, NC), jnp.float32),
            jax.ShapeDtypeStruct((B, N, 4), jnp.float32),
        ],
    )(prediction)


def _stage_b_kernel(scores_hbm, vals_hbm, idx_hbm, chunk_v, valbuf, idxbuf):
    wid = lax.axis_index("s") * 2 + lax.axis_index("c")
    img = wid // TPI
    sub = wid % TPI
    base = sub * RANGE

    def init_body(i, _):
        valbuf[pl.ds(i * 16, 16)] = jnp.full((16,), -1.0, jnp.float32)
        idxbuf[pl.ds(i * 16, 16)] = jnp.zeros((16,), jnp.int32)
        return 0

    lax.fori_loop(0, CT // 16, init_body, 0)

    lane = lax.iota(jnp.int32, 16)

    def chunk_body(ci, cnt):
        pltpu.sync_copy(
            scores_hbm.at[pl.ds(img * NFLAT + base + ci * CHUNK, CHUNK)],
            chunk_v)

        def vec_body(vi, cnt_in):
            vs = [chunk_v[pl.ds(vi * 64 + 16 * g, 16)] for g in range(4)]
            ms = [v > 0.0 for v in vs]
            mis = [m.astype(jnp.int32) for m in ms]
            tot = jnp.sum(mis[0] + mis[1] + mis[2] + mis[3])

            def compact(c):
                for g in range(4):
                    gidx = lane + (base + ci * CHUNK + vi * 64 + 16 * g)
                    pos = jnp.minimum(c + plsc.cumsum(mis[g]) - mis[g],
                                      CT - 1)
                    plsc.store_scatter(valbuf, [pos], vs[g], mask=ms[g])
                    plsc.store_scatter(idxbuf, [pos], gidx, mask=ms[g])
                    c = c + jnp.sum(mis[g])
                return c

            return lax.cond(tot > 0, compact, lambda c: c, cnt_in)

        return lax.fori_loop(0, CHUNK // 64, vec_body, cnt)

    lax.fori_loop(0, RANGE // CHUNK, chunk_body, jnp.int32(0))

    pltpu.sync_copy(valbuf, vals_hbm.at[pl.ds(img * SORT_N + sub * CT, CT)])
    pltpu.sync_copy(idxbuf, idx_hbm.at[pl.ds(img * SORT_N + sub * CT, CT)])


def _stage_b(scores_flat):
    mesh = plsc.VectorSubcoreMesh(core_axis_name="c", subcore_axis_name="s")
    k = functools.partial(
        pl.kernel,
        mesh=mesh,
        compiler_params=pltpu.CompilerParams(needs_layout_passes=False),
        out_type=[
            jax.ShapeDtypeStruct((B * SORT_N,), jnp.float32),
            jax.ShapeDtypeStruct((B * SORT_N,), jnp.int32),
        ],
        scratch_types=[
            pltpu.VMEM((CHUNK,), jnp.float32),
            pltpu.VMEM((CT,), jnp.float32),
            pltpu.VMEM((CT,), jnp.int32),
        ],
    )(_stage_b_kernel)
    vals, idxs = k(scores_flat.reshape(B * NFLAT))
    return vals.reshape(B, SORT_N), idxs.reshape(B, SORT_N)


def _roll(x, shift):
    return jnp.concatenate([x[..., -shift % 128:], x[..., : -shift % 128]],
                           axis=-1)


def _stage_c_body(val_ref, idx_ref, ts_ref, bi_ref, cf_ref):
    s = val_ref[...]
    ix = idx_ref[...]
    r_io = lax.broadcasted_iota(jnp.int32, (B, 128, 128), 1)
    c_io = lax.broadcasted_iota(jnp.int32, (B, 128, 128), 2)

    n_total = 128 * 128
    k = 2
    while k <= n_total:
        j = k // 2
        while j >= 1:
            if j >= 128:
                rj = j // 128
                sh = (B, 128 // (2 * rj), 2, rj, 128)
                s4 = s.reshape(sh)
                i4 = ix.reshape(sh)
                s_p = jnp.concatenate([s4[:, :, 1:2], s4[:, :, 0:1]],
                                      axis=2).reshape(B, 128, 128)
                i_p = jnp.concatenate([i4[:, :, 1:2], i4[:, :, 0:1]],
                                      axis=2).reshape(B, 128, 128)
                am_hi = (r_io & rj) != 0
            else:
                s_m = _roll(s, -j)
                s_q = _roll(s, j)
                i_m = _roll(ix, -j)
                i_q = _roll(ix, j)
                lane_lo = (c_io & j) == 0
                s_p = jnp.where(lane_lo, s_m, s_q)
                i_p = jnp.where(lane_lo, i_m, i_q)
                am_hi = ~lane_lo
            if k >= 128:
                asc = (r_io & (k // 128)) == 0
            else:
                asc = (c_io & k) == 0
            take_high = asc != am_hi
            mine_high = (s > s_p) | ((s == s_p) & (ix < i_p))
            sel_mine = ~(take_high ^ mine_high)
            s = jnp.where(sel_mine, s, s_p)
            ix = jnp.where(sel_mine, ix, i_p)
            j //= 2
        k *= 2

    ts = s[:, :16, :]
    ti = ix[:, :16, :]
    ts_ref[...] = ts
    ti_f = ti.astype(jnp.float32)
    bi_f = jnp.floor((ti_f + 0.5) * (1.0 / NC))
    bi_ref[...] = bi_f.astype(jnp.int32)
    cf_ref[...] = ti_f - bi_f * float(NC)


def _stage_c(vals, idxs):
    v3 = vals.reshape(B, 128, 128)
    i3 = idxs.reshape(B, 128, 128)
    ts, bi, cf = pl.pallas_call(
        _stage_c_body,
        out_shape=[
            jax.ShapeDtypeStruct((B, 16, 128), jnp.float32),
            jax.ShapeDtypeStruct((B, 16, 128), jnp.int32),
            jax.ShapeDtypeStruct((B, 16, 128), jnp.float32),
        ],
    )(v3, i3)
    return (ts.reshape(B, PRE_NMS), bi.reshape(B, PRE_NMS),
            cf.reshape(B, PRE_NMS))


def _stage_d_kernel(boxes_hbm, bidx_hbm, out_hbm, table_v, idx_v, ob):
    wid = lax.axis_index("s") * 2 + lax.axis_index("c")
    img = wid // TPI
    sub = wid % TPI
    per = PRE_NMS // TPI

    pltpu.sync_copy(boxes_hbm.at[pl.ds(img * N * 4, N * 4)], table_v)
    pltpu.sync_copy(bidx_hbm.at[pl.ds(img * PRE_NMS + sub * per, per)], idx_v)

    def body(g, _):
        iv = idx_v[pl.ds(g * 16, 16)]
        for c in range(4):
            v = plsc.load_gather(table_v, [iv * 4 + c])
            ob[c, pl.ds(g * 16, 16)] = v
        return 0

    lax.fori_loop(0, per // 16, body, 0)
    for c in range(4):
        pltpu.sync_copy(
            ob.at[c],
            out_hbm.at[pl.ds(img * 4 * PRE_NMS + c * PRE_NMS + sub * per,
                             per)])


def _stage_d(boxes_flat, bidx):
    mesh = plsc.VectorSubcoreMesh(core_axis_name="c", subcore_axis_name="s")
    per = PRE_NMS // TPI
    k = functools.partial(
        pl.kernel,
        mesh=mesh,
        compiler_params=pltpu.CompilerParams(needs_layout_passes=False),
        out_type=jax.ShapeDtypeStruct((B * 4 * PRE_NMS,), jnp.float32),
        scratch_types=[
            pltpu.VMEM((N * 4,), jnp.float32),
            pltpu.VMEM((per,), jnp.int32),
            pltpu.VMEM((4, per), jnp.float32),
        ],
    )(_stage_d_kernel)
    return k(boxes_flat.reshape(B * N * 4),
             bidx.reshape(B * PRE_NMS)).reshape(B, 4, PRE_NMS)


def _stage_e_body(boxp_ref, cls_ref, sc_ref, keep_ref, s_scr):
    cls = cls_ref[...]
    off = cls * MAX_WH
    x1 = boxp_ref[:, 0, :] + off
    y1 = boxp_ref[:, 1, :] + off
    x2 = boxp_ref[:, 2, :] + off
    y2 = boxp_ref[:, 3, :] + off
    area = (x2 - x1) * (y2 - y1)
    valid = jnp.where(sc_ref[...] > 0.0, 1.0, 0.0)

    lane128 = lax.broadcasted_iota(jnp.int32, (1, 128), 1)
    supp = jnp.zeros((B, PRE_NMS), jnp.float32)
    keeps = []
    for b in range(16):
        lo = b * 128
        W = PRE_NMS - lo
        bx1 = x1[:, lo:lo + 128]
        by1 = y1[:, lo:lo + 128]
        bx2 = x2[:, lo:lo + 128]
        by2 = y2[:, lo:lo + 128]
        barea = area[:, lo:lo + 128]
        tx1 = x1[:, lo:]
        ty1 = y1[:, lo:]
        tx2 = x2[:, lo:]
        ty2 = y2[:, lo:]
        tarea = area[:, lo:]
        ltx = jnp.maximum(bx1[:, :, None], tx1[:, None, :])
        lty = jnp.maximum(by1[:, :, None], ty1[:, None, :])
        rbx = jnp.minimum(bx2[:, :, None], tx2[:, None, :])
        rby = jnp.minimum(by2[:, :, None], ty2[:, None, :])
        iw = jnp.clip(rbx - ltx, 0.0, None)
        ih = jnp.clip(rby - lty, 0.0, None)
        inter = iw * ih
        den = barea[:, :, None] + tarea[:, None, :] - inter + 1e-9
        iou = inter / den
        s_scr[:, :, 0:W] = jnp.where(iou > IOU_THRES, 1.0, 0.0)

        valid_b = valid[:, lo:lo + 128]
        supp_b0 = supp[:, lo:lo + 128]
        free_b0 = valid_b * (1.0 - supp_b0)

        def body(i8, st):
            free_b, keep_b = st
            bs = pl.multiple_of(i8 * 8, 8)
            s8 = s_scr[:, pl.ds(bs, 8), 0:128]
            for h in range(2):
                i0 = i8 * 8 + h * 4
                e = [jnp.where(lane128 == i0 + q, 1.0, 0.0) for q in range(4)]
                sr = [s8[:, h * 4 + q, :] for q in range(4)]
                ra = [jnp.max(e[q] * free_b, axis=-1, keepdims=True)
                      for q in range(4)]
                sij = {}
                for a in range(4):
                    for bq in range(a + 1, 4):
                        sij[(a, bq)] = jnp.max(e[bq] * sr[a], axis=-1,
                                               keepdims=True)
                r1 = ra[0]
                r2 = ra[1] * (1.0 - r1 * sij[(0, 1)])
                r3 = ra[2] * (1.0 - r1 * sij[(0, 2)]) * (1.0 - r2 * sij[(1, 2)])
                r4 = (ra[3] * (1.0 - r1 * sij[(0, 3)])
                      * (1.0 - r2 * sij[(1, 3)]) * (1.0 - r3 * sij[(2, 3)]))
                free_b = free_b * ((1.0 - r1 * sr[0]) * (1.0 - r2 * sr[1])
                                   * ((1.0 - r3 * sr[2]) * (1.0 - r4 * sr[3])))
                keep_b = keep_b + (e[0] * r1 + e[1] * r2
                                   + (e[2] * r3 + e[3] * r4))
            return free_b, keep_b

        keep_b0 = jnp.zeros((B, 128), jnp.float32)
        _, keep_bf = lax.fori_loop(0, 16, body, (free_b0, keep_b0))
        keeps.append(keep_bf)

        s_tail = s_scr[:, :, 0:W]
        prop = jnp.max(keep_bf[:, :, None] * s_tail, axis=1)
        if lo > 0:
            prop = jnp.concatenate(
                [jnp.zeros((B, lo), jnp.float32), prop], axis=1)
        supp = jnp.maximum(supp, prop)

    keep_ref[...] = jnp.concatenate(keeps, axis=1)


def _stage_e(boxesP, cls_f, top_scores):
    return pl.pallas_call(
        _stage_e_body,
        out_shape=jax.ShapeDtypeStruct((B, PRE_NMS), jnp.float32),
        scratch_shapes=[pltpu.VMEM((B, 128, PRE_NMS), jnp.float32)],
    )(boxesP, cls_f, top_scores)


BUFW = 1808


def _stage_f_kernel(keep_hbm, sc_hbm, cls_hbm, boxp_hbm, out_hbm,
                    keep_v, sc_v, cls_v, box_v, buf):
    wid = lax.axis_index("s") * 2 + lax.axis_index("c")

    @pl.when(wid < B)
    def _():
        img = wid

        def zero_body(i, _):
            buf[pl.ds(i * 16, 16)] = jnp.zeros((16,), jnp.float32)
            return 0

        lax.fori_loop(0, BUFW // 16, zero_body, 0)

        pltpu.sync_copy(keep_hbm.at[pl.ds(img * PRE_NMS, PRE_NMS)], keep_v)
        pltpu.sync_copy(sc_hbm.at[pl.ds(img * PRE_NMS, PRE_NMS)], sc_v)
        pltpu.sync_copy(cls_hbm.at[pl.ds(img * PRE_NMS, PRE_NMS)], cls_v)
        pltpu.sync_copy(boxp_hbm.at[pl.ds(img * 4 * PRE_NMS, 4 * PRE_NMS)],
                        box_v)

        def body(g, cnt):
            km = keep_v[pl.ds(g * 16, 16)] > 0.5
            ki = km.astype(jnp.int32)
            pos = plsc.cumsum(ki) - ki + cnt
            pos = jnp.minimum(pos, MAX_DET)
            mw = km & (pos < MAX_DET)
            base6 = pos * 6
            for f in range(6):
                if f < 4:
                    v = box_v[pl.ds(f * PRE_NMS + g * 16, 16)]
                elif f == 4:
                    v = sc_v[pl.ds(g * 16, 16)]
                else:
                    v = cls_v[pl.ds(g * 16, 16)]
                plsc.store_scatter(buf, [base6 + f], v, mask=mw)
            return cnt + jnp.sum(ki)

        lax.fori_loop(0, PRE_NMS // 16, body, jnp.int32(0))
        pltpu.sync_copy(buf.at[pl.ds(0, OUTW)],
                        out_hbm.at[pl.ds(img * OUTW, OUTW)])


def _stage_f(keep, top_scores, cls_f, boxesP):
    mesh = plsc.VectorSubcoreMesh(core_axis_name="c", subcore_axis_name="s")
    k = functools.partial(
        pl.kernel,
        mesh=mesh,
        compiler_params=pltpu.CompilerParams(needs_layout_passes=False),
        out_type=jax.ShapeDtypeStruct((B * OUTW,), jnp.float32),
        scratch_types=[
            pltpu.VMEM((PRE_NMS,), jnp.float32),
            pltpu.VMEM((PRE_NMS,), jnp.float32),
            pltpu.VMEM((PRE_NMS,), jnp.float32),
            pltpu.VMEM((4 * PRE_NMS,), jnp.float32),
            pltpu.VMEM((BUFW,), jnp.float32),
        ],
    )(_stage_f_kernel)
    return k(keep.reshape(B * PRE_NMS), top_scores.reshape(B * PRE_NMS),
             cls_f.reshape(B * PRE_NMS),
             boxesP.reshape(B * 4 * PRE_NMS)).reshape(B, OUTW)


def kernel(prediction):
    scores, boxes = _stage_a(prediction)
    vals, idxs = _stage_b(scores.reshape(B, NFLAT))
    ts, bi, cf = _stage_c(vals, idxs)
    boxesP = _stage_d(boxes.reshape(B, N * 4), bi)
    keep = _stage_e(boxesP, cf, ts)
    out = _stage_f(keep, ts, cf, boxesP)
    return out.reshape(B, MAX_DET, 6)
